# baseline (speedup 1.0000x reference)
import functools

import jax
import jax.numpy as jnp
from jax import lax
from jax.experimental import pallas as pl
from jax.experimental.pallas import tpu as pltpu
from jax.experimental.pallas import tpu_sc as plsc

HIDDEN_DIM = 2048
DELTA = 1.0 / HIDDEN_DIM
GAMMA = 0.05
LAMBDA_KL = 0.01
BETA = 0.05

ROWS = 4 * 8192
BLOCK_ROWS = 1024
NSTEPS = ROWS // BLOCK_ROWS
CHUNK = 2048
GROUP = 128
NGROUPS = HIDDEN_DIM // GROUP
NV = HIDDEN_DIM // 16


def _main_body(x_ref, h_ref, csum_ref, kl_ref, acc_ref, cnt_ref):
    i = pl.program_id(0)
    xb = x_ref[...]
    h_ref[...] = xb
    relu = jnp.maximum(xb, 0.0)
    psum = jnp.sum(relu, axis=0, keepdims=True)
    pcnt = jnp.sum((xb > 0.0).astype(jnp.float32), axis=0, keepdims=True)

    @pl.when(i == 0)
    def _():
        acc_ref[...] = psum
        cnt_ref[...] = pcnt

    @pl.when(i > 0)
    def _():
        acc_ref[...] += psum
        cnt_ref[...] += pcnt

    @pl.when(i == NSTEPS - 1)
    def _():
        csum_ref[...] = acc_ref[...]
        rho = jnp.sum(cnt_ref[...]) * (1.0 / (ROWS * HIDDEN_DIM))
        rho = jnp.clip(rho, 1e-05, 1.0 - 1e-05)
        kl_ref[0, 0] = LAMBDA_KL * (
            rho * jnp.log(rho / BETA)
            + (1.0 - rho) * jnp.log((1.0 - rho) / (1.0 - BETA)))


def _ds16(i):
    return pl.ds(pl.multiple_of(i * 16, 16), 16)


def _sc_epi_body(cs_hbm, e_hbm, n_hbm,
                 ne_hbm, msk_hbm, val_hbm, pen_hbm,
                 cs_v, e_v, n_v, ne_v, msk_v, val_v, pen_v, sem):
    wid = lax.axis_index("s") * 2 + lax.axis_index("c")

    @pl.when(wid == 0)
    def _():
        pltpu.make_async_copy(cs_hbm, cs_v, sem).start()
        pltpu.make_async_copy(cs_hbm, cs_v, sem).wait()
        pltpu.make_async_copy(e_hbm, e_v, sem).start()
        pltpu.make_async_copy(e_hbm, e_v, sem).wait()
        pltpu.make_async_copy(n_hbm, n_v, sem).start()
        pltpu.make_async_copy(n_hbm, n_v, sem).wait()

        def step(i, pen):
            e = e_v[_ds16(i)]
            ne = e + DELTA + n_v[_ds16(i)] - (GAMMA / ROWS) * cs_v[_ds16(i)]
            a = jnp.abs(ne)
            pen = jnp.where(a > 1.0, pen + (a - 1.0), pen)
            fire = ne >= 2.0
            shut = ne <= -2.0
            ne_v[_ds16(i)] = jnp.where(shut, -2.0, ne)
            m = jnp.logical_or(fire, shut)
            msk_v[_ds16(i)] = jnp.where(m, 1.0, 0.0)
            val_v[_ds16(i)] = jnp.where(shut, e + 2.0, 2.0)
            return pen

        pen = lax.fori_loop(0, NV, step, jnp.zeros((16,), jnp.float32))
        pen_v[...] = pen

        pltpu.make_async_copy(ne_v, ne_hbm, sem).start()
        pltpu.make_async_copy(ne_v, ne_hbm, sem).wait()
        pltpu.make_async_copy(msk_v, msk_hbm, sem).start()
        pltpu.make_async_copy(msk_v, msk_hbm, sem).wait()
        pltpu.make_async_copy(val_v, val_hbm, sem).start()
        pltpu.make_async_copy(val_v, val_hbm, sem).wait()
        pltpu.make_async_copy(pen_v, pen_hbm, sem).start()
        pltpu.make_async_copy(pen_v, pen_hbm, sem).wait()


def _sc_epilogue(csum, energy, noise):
    mesh = plsc.VectorSubcoreMesh(core_axis_name="c", subcore_axis_name="s")
    f = functools.partial(
        pl.kernel,
        out_type=[
            jax.ShapeDtypeStruct((HIDDEN_DIM,), jnp.float32),
            jax.ShapeDtypeStruct((HIDDEN_DIM,), jnp.float32),
            jax.ShapeDtypeStruct((HIDDEN_DIM,), jnp.float32),
            jax.ShapeDtypeStruct((16,), jnp.float32),
        ],
        mesh=mesh,
        scratch_types=[
            pltpu.VMEM((HIDDEN_DIM,), jnp.float32),
            pltpu.VMEM((HIDDEN_DIM,), jnp.float32),
            pltpu.VMEM((HIDDEN_DIM,), jnp.float32),
            pltpu.VMEM((HIDDEN_DIM,), jnp.float32),
            pltpu.VMEM((HIDDEN_DIM,), jnp.float32),
            pltpu.VMEM((HIDDEN_DIM,), jnp.float32),
            pltpu.VMEM((16,), jnp.float32),
            pltpu.SemaphoreType.DMA,
        ],
    )(_sc_epi_body)
    return f(csum, energy, noise)


def _fix_body(h_in_ref, msk_ref, val_ref, h_ref, buf, sem):
    del h_in_ref
    for g in range(NGROUPS):
        mg = msk_ref[0:1, g * GROUP:(g + 1) * GROUP]

        @pl.when(jnp.sum(mg) > 0.0)
        def _():
            vg = val_ref[0:1, g * GROUP:(g + 1) * GROUP]

            def per_chunk(r, c):
                stripe = h_ref.at[pl.ds(r * CHUNK, CHUNK),
                                  pl.ds(g * GROUP, GROUP)]
                pltpu.make_async_copy(stripe, buf, sem).start()
                pltpu.make_async_copy(stripe, buf, sem).wait()
                buf[...] = jnp.where(mg > 0.5, vg, buf[...])
                pltpu.make_async_copy(buf, stripe, sem).start()
                pltpu.make_async_copy(buf, stripe, sem).wait()
                return c

            lax.fori_loop(0, ROWS // CHUNK, per_chunk, 0)


@jax.jit
def kernel(x, energy, noise):
    xf = x.reshape(ROWS, HIDDEN_DIM)

    h0, csum, kl = pl.pallas_call(
        _main_body,
        grid=(NSTEPS,),
        in_specs=[
            pl.BlockSpec((BLOCK_ROWS, HIDDEN_DIM), lambda i: (i, 0)),
        ],
        out_specs=[
            pl.BlockSpec((BLOCK_ROWS, HIDDEN_DIM), lambda i: (i, 0)),
            pl.BlockSpec((1, HIDDEN_DIM), lambda i: (0, 0)),
            pl.BlockSpec((1, 1), lambda i: (0, 0), memory_space=pltpu.SMEM),
        ],
        out_shape=[
            jax.ShapeDtypeStruct((ROWS, HIDDEN_DIM), jnp.float32),
            jax.ShapeDtypeStruct((1, HIDDEN_DIM), jnp.float32),
            jax.ShapeDtypeStruct((1, 1), jnp.float32),
        ],
        scratch_shapes=[
            pltpu.VMEM((1, HIDDEN_DIM), jnp.float32),
            pltpu.VMEM((1, HIDDEN_DIM), jnp.float32),
        ],
        compiler_params=pltpu.CompilerParams(
            dimension_semantics=("arbitrary",),
        ),
    )(xf)

    ne, msk, val, pen = _sc_epilogue(csum.reshape(HIDDEN_DIM), energy, noise)

    h = pl.pallas_call(
        _fix_body,
        in_specs=[
            pl.BlockSpec(memory_space=pl.ANY),
            pl.BlockSpec(memory_space=pltpu.VMEM),
            pl.BlockSpec(memory_space=pltpu.VMEM),
        ],
        out_specs=pl.BlockSpec(memory_space=pl.ANY),
        out_shape=jax.ShapeDtypeStruct((ROWS, HIDDEN_DIM), jnp.float32),
        scratch_shapes=[
            pltpu.VMEM((CHUNK, GROUP), jnp.float32),
            pltpu.SemaphoreType.DMA,
        ],
        input_output_aliases={0: 0},
    )(h0, msk.reshape(1, HIDDEN_DIM), val.reshape(1, HIDDEN_DIM))

    aux = kl[0, 0] + 0.01 * jnp.sum(pen)
    return (h.reshape(x.shape), ne, aux)


# speedup vs baseline: 1.1457x; 1.1457x over previous
import functools

import jax
import jax.numpy as jnp
from jax import lax
from jax.experimental import pallas as pl
from jax.experimental.pallas import tpu as pltpu

HIDDEN_DIM = 2048
DELTA = 1.0 / HIDDEN_DIM
GAMMA = 0.05
LAMBDA_KL = 0.01
BETA = 0.05

ROWS = 4 * 8192
BLOCK_ROWS = 1024
NSTEPS = ROWS // BLOCK_ROWS
CHUNK = 2048
GROUP = 128
NGROUPS = HIDDEN_DIM // GROUP


def _main_body(x_ref, e_ref, n_ref,
               h_ref, ne_ref, aux_ref, msk_ref, val_ref, gcnt_ref,
               acc_ref, cnt_ref):
    i = pl.program_id(0)
    xb = x_ref[...]
    h_ref[...] = xb
    relu = jnp.maximum(xb, 0.0)
    psum = jnp.sum(relu, axis=0, keepdims=True)
    pcnt = jnp.sum((xb > 0.0).astype(jnp.float32), axis=0, keepdims=True)

    @pl.when(i == 0)
    def _():
        acc_ref[...] = psum
        cnt_ref[...] = pcnt

    @pl.when(i > 0)
    def _():
        acc_ref[...] += psum
        cnt_ref[...] += pcnt

    @pl.when(i == NSTEPS - 1)
    def _():
        colmean = acc_ref[...] * (1.0 / ROWS)
        e = e_ref[...]
        ne = e + DELTA + n_ref[...] - GAMMA * colmean
        rho = jnp.sum(cnt_ref[...]) * (1.0 / (ROWS * HIDDEN_DIM))
        rho = jnp.clip(rho, 1e-05, 1.0 - 1e-05)
        kl = LAMBDA_KL * (rho * jnp.log(rho / BETA)
                          + (1.0 - rho) * jnp.log((1.0 - rho) / (1.0 - BETA)))
        high = ne > 1.0
        low = ne < -1.0
        pen = (0.01 * jnp.sum(jnp.where(high, jnp.abs(ne) - 1.0, 0.0))
               + 0.01 * jnp.sum(jnp.where(low, jnp.abs(ne) - 1.0, 0.0)))
        aux_ref[0, 0] = kl + pen
        fire = ne >= 2.0
        shut = ne <= -2.0
        ne_ref[...] = jnp.where(shut, -2.0, ne)
        m = jnp.logical_or(fire, shut)
        mi = m.astype(jnp.int32)
        msk_ref[...] = m.astype(jnp.float32)
        val_ref[...] = jnp.where(shut, e + 2.0, 2.0)
        for k in range(NGROUPS):
            gcnt_ref[0, k] = jnp.sum(mi[0, k * GROUP:(k + 1) * GROUP])


def _fix_body(h_in_ref, msk_ref, val_ref, gcnt_ref, h_ref, buf, sem):
    del h_in_ref
    for g in range(NGROUPS):
        @pl.when(gcnt_ref[0, g] > 0)
        def _():
            mg = msk_ref[0:1, g * GROUP:(g + 1) * GROUP] > 0.5
            vg = val_ref[0:1, g * GROUP:(g + 1) * GROUP]

            def per_chunk(r, c):
                stripe = h_ref.at[pl.ds(r * CHUNK, CHUNK),
                                  pl.ds(g * GROUP, GROUP)]
                pltpu.make_async_copy(stripe, buf, sem).start()
                pltpu.make_async_copy(stripe, buf, sem).wait()
                buf[...] = jnp.where(mg, vg, buf[...])
                pltpu.make_async_copy(buf, stripe, sem).start()
                pltpu.make_async_copy(buf, stripe, sem).wait()
                return c

            lax.fori_loop(0, ROWS // CHUNK, per_chunk, 0)


@jax.jit
def kernel(x, energy, noise):
    xf = x.reshape(ROWS, HIDDEN_DIM)
    e2 = energy.reshape(1, HIDDEN_DIM)
    n2 = noise.reshape(1, HIDDEN_DIM)

    h0, ne, aux, msk, val, gcnt = pl.pallas_call(
        _main_body,
        grid=(NSTEPS,),
        in_specs=[
            pl.BlockSpec((BLOCK_ROWS, HIDDEN_DIM), lambda i: (i, 0)),
            pl.BlockSpec((1, HIDDEN_DIM), lambda i: (0, 0)),
            pl.BlockSpec((1, HIDDEN_DIM), lambda i: (0, 0)),
        ],
        out_specs=[
            pl.BlockSpec((BLOCK_ROWS, HIDDEN_DIM), lambda i: (i, 0)),
            pl.BlockSpec((1, HIDDEN_DIM), lambda i: (0, 0)),
            pl.BlockSpec((1, 1), lambda i: (0, 0), memory_space=pltpu.SMEM),
            pl.BlockSpec((1, HIDDEN_DIM), lambda i: (0, 0)),
            pl.BlockSpec((1, HIDDEN_DIM), lambda i: (0, 0)),
            pl.BlockSpec((1, NGROUPS), lambda i: (0, 0), memory_space=pltpu.SMEM),
        ],
        out_shape=[
            jax.ShapeDtypeStruct((ROWS, HIDDEN_DIM), jnp.float32),
            jax.ShapeDtypeStruct((1, HIDDEN_DIM), jnp.float32),
            jax.ShapeDtypeStruct((1, 1), jnp.float32),
            jax.ShapeDtypeStruct((1, HIDDEN_DIM), jnp.float32),
            jax.ShapeDtypeStruct((1, HIDDEN_DIM), jnp.float32),
            jax.ShapeDtypeStruct((1, NGROUPS), jnp.int32),
        ],
        scratch_shapes=[
            pltpu.VMEM((1, HIDDEN_DIM), jnp.float32),
            pltpu.VMEM((1, HIDDEN_DIM), jnp.float32),
        ],
        compiler_params=pltpu.CompilerParams(
            dimension_semantics=("arbitrary",),
        ),
    )(xf, e2, n2)

    h = pl.pallas_call(
        _fix_body,
        in_specs=[
            pl.BlockSpec(memory_space=pl.ANY),
            pl.BlockSpec(memory_space=pltpu.VMEM),
            pl.BlockSpec(memory_space=pltpu.VMEM),
            pl.BlockSpec(memory_space=pltpu.SMEM),
        ],
        out_specs=pl.BlockSpec(memory_space=pl.ANY),
        out_shape=jax.ShapeDtypeStruct((ROWS, HIDDEN_DIM), jnp.float32),
        scratch_shapes=[
            pltpu.VMEM((CHUNK, GROUP), jnp.float32),
            pltpu.SemaphoreType.DMA,
        ],
        input_output_aliases={0: 0},
    )(h0, msk, val, gcnt)

    return (h.reshape(x.shape), ne.reshape(HIDDEN_DIM), aux[0, 0])
